# Initial kernel scaffold; baseline (speedup 1.0000x reference)
import jax
import jax.numpy as jnp
from jax.experimental import pallas as pl


def kernel(predicted_locs, predicted_scores, boxes, labels, priors_cxcy):
    raise NotImplementedError("write your pallas kernel here")


# speedup vs baseline: 20.4500x; 20.4500x over previous
import functools

import jax
import jax.numpy as jnp
from jax.experimental import pallas as pl
from jax.experimental.pallas import tpu as pltpu

ROWS = 192
LANES = 128
P_PAD = ROWS * LANES
NUM_C = 21
NUM_O = 16
NEG_POS_RATIO = 3
OVERLAP_THRESHOLD = 0.5


def _mbl_kernel(boxes_ref, labels_ref, priors_ref, scores_ref, locs_ref,
                out_ref, *, n_valid):
    f32 = jnp.float32
    i32 = jnp.int32
    px = priors_ref[0]
    py = priors_ref[1]
    pw = priors_ref[2]
    ph = priors_ref[3]
    pxl = px - pw * 0.5
    pxh = px + pw * 0.5
    pyl = py - ph * 0.5
    pyh = py + ph * 0.5
    p_area = pw * ph

    row = jax.lax.broadcasted_iota(i32, (ROWS, LANES), 0)
    lane = jax.lax.broadcasted_iota(i32, (ROWS, LANES), 1)
    flat = row * LANES + lane
    valid = flat < n_valid

    best_ov = jnp.zeros((ROWS, LANES), f32)
    best_obj = jnp.zeros((ROWS, LANES), i32)
    big = i32(2**30)
    pfo = []
    for o in range(NUM_O):
        bxl = boxes_ref[0, o, 0]
        byl = boxes_ref[0, o, 1]
        bxh = boxes_ref[0, o, 2]
        byh = boxes_ref[0, o, 3]
        b_area = (bxh - bxl) * (byh - byl)
        iw = jnp.maximum(jnp.minimum(pxh, bxh) - jnp.maximum(pxl, bxl), 0.0)
        ih = jnp.maximum(jnp.minimum(pyh, byh) - jnp.maximum(pyl, byl), 0.0)
        inter = iw * ih
        ov = inter / (p_area + b_area - inter)
        ov = jnp.where(valid, ov, -1.0)
        upd = ov > best_ov
        best_ov = jnp.where(upd, ov, best_ov)
        best_obj = jnp.where(upd, o, best_obj)
        m = jnp.max(ov)
        pfo.append(jnp.min(jnp.where(ov == m, flat, big)))
    for o in range(NUM_O):
        hit = flat == pfo[o]
        best_obj = jnp.where(hit, o, best_obj)
        best_ov = jnp.where(hit, 1.0, best_ov)

    lab = jnp.zeros((ROWS, LANES), i32)
    gxl = jnp.zeros((ROWS, LANES), f32)
    gyl = jnp.zeros((ROWS, LANES), f32)
    gxh = jnp.zeros((ROWS, LANES), f32)
    gyh = jnp.zeros((ROWS, LANES), f32)
    for o in range(NUM_O):
        m = best_obj == o
        lab = jnp.where(m, labels_ref[0, 0, o], lab)
        gxl = jnp.where(m, boxes_ref[0, o, 0], gxl)
        gyl = jnp.where(m, boxes_ref[0, o, 1], gyl)
        gxh = jnp.where(m, boxes_ref[0, o, 2], gxh)
        gyh = jnp.where(m, boxes_ref[0, o, 3], gyh)
    lab = jnp.where(best_ov < OVERLAP_THRESHOLD, 0, lab)
    pos = lab != 0
    posf = pos.astype(f32)
    n_pos_i = jnp.sum(pos.astype(i32))

    gcx = (gxl + gxh) * 0.5
    gcy = (gyl + gyh) * 0.5
    gw = gxh - gxl
    gh = gyh - gyl
    t0 = (gcx - px) * 10.0 / pw
    t1 = (gcy - py) * 10.0 / ph
    t2 = jnp.log(gw / pw) * 5.0
    t3 = jnp.log(gh / ph) * 5.0
    hub = jnp.zeros((), f32)
    for c, t in enumerate((t0, t1, t2, t3)):
        d = locs_ref[0, c] - t
        ad = jnp.abs(d)
        h = jnp.where(ad < 1.0, 0.5 * d * d, ad - 0.5)
        hub = hub + jnp.sum(h * posf)

    mx = scores_ref[0, 0]
    for c in range(1, NUM_C):
        mx = jnp.maximum(mx, scores_ref[0, c])
    se = jnp.zeros((ROWS, LANES), f32)
    st = jnp.zeros((ROWS, LANES), f32)
    for c in range(NUM_C):
        s = scores_ref[0, c]
        se = se + jnp.exp(s - mx)
        st = jnp.where(lab == c, s, st)
    conf = mx + jnp.log(se) - st
    conf_pos_sum = jnp.sum(conf * posf)
    vneg = jnp.where(pos | jnp.logical_not(valid), 0.0, conf)

    k = NEG_POS_RATIO * n_pos_i
    vbits = jax.lax.bitcast_convert_type(vneg, i32)

    def body(_, lohi):
        lo, hi = lohi
        mid = lo + (hi - lo) // 2
        cnt = jnp.sum((vbits >= mid).astype(i32))
        return jnp.where(cnt >= k, mid, lo), jnp.where(cnt >= k, hi, mid)

    lo, hi = jax.lax.fori_loop(0, 31, body, (i32(0), i32(2**31 - 1)))
    gt = vbits > lo
    sum_gt = jnp.sum(jnp.where(gt, vneg, 0.0))
    cnt_gt = jnp.sum(gt.astype(i32))
    vk = jnp.max(jnp.where(vbits == lo, vneg, -1.0))
    hard_neg_sum = sum_gt + (k - cnt_gt).astype(f32) * vk

    out_ref[0, 0] = jnp.full((8, LANES), hub, f32)
    out_ref[0, 1] = jnp.full((8, LANES), n_pos_i.astype(f32), f32)
    out_ref[0, 2] = jnp.full((8, LANES), conf_pos_sum, f32)
    out_ref[0, 3] = jnp.full((8, LANES), hard_neg_sum, f32)


def kernel(predicted_locs, predicted_scores, boxes, labels, priors_cxcy):
    B, P, C = predicted_scores.shape
    pad = P_PAD - P
    scores_t = jnp.transpose(predicted_scores, (0, 2, 1))
    scores_t = jnp.pad(scores_t, ((0, 0), (0, 0), (0, pad)))
    scores_t = scores_t.reshape(B, C, ROWS, LANES)
    locs_t = jnp.transpose(predicted_locs, (0, 2, 1))
    locs_t = jnp.pad(locs_t, ((0, 0), (0, 0), (0, pad)))
    locs_t = locs_t.reshape(B, 4, ROWS, LANES)
    priors_t = jnp.pad(jnp.transpose(priors_cxcy, (1, 0)), ((0, 0), (0, pad)),
                       constant_values=1.0)
    priors_t = priors_t.reshape(4, ROWS, LANES)
    labels3 = labels.astype(jnp.int32).reshape(B, 1, NUM_O)
    boxes = boxes.astype(jnp.float32)

    out = pl.pallas_call(
        functools.partial(_mbl_kernel, n_valid=P),
        grid=(B,),
        in_specs=[
            pl.BlockSpec((1, NUM_O, 4), lambda b: (b, 0, 0)),
            pl.BlockSpec((1, 1, NUM_O), lambda b: (b, 0, 0)),
            pl.BlockSpec((4, ROWS, LANES), lambda b: (0, 0, 0)),
            pl.BlockSpec((1, NUM_C, ROWS, LANES), lambda b: (b, 0, 0, 0)),
            pl.BlockSpec((1, 4, ROWS, LANES), lambda b: (b, 0, 0, 0)),
        ],
        out_specs=pl.BlockSpec((1, 4, 8, LANES), lambda b: (b, 0, 0, 0)),
        out_shape=jax.ShapeDtypeStruct((B, 4, 8, LANES), jnp.float32),
    )(boxes, labels3, priors_t, scores_t, locs_t)

    hub = jnp.sum(out[:, 0, 0, 0])
    n_pos_total = jnp.sum(out[:, 1, 0, 0])
    conf_pos = jnp.sum(out[:, 2, 0, 0])
    hard_neg = jnp.sum(out[:, 3, 0, 0])
    conf_loss = (conf_pos + hard_neg) / n_pos_total
    loc_loss = hub / (4.0 * n_pos_total)
    return conf_loss + loc_loss
